# Initial kernel scaffold; baseline (speedup 1.0000x reference)
import jax
import jax.numpy as jnp
from jax.experimental import pallas as pl


def kernel(net_feat, net_edge_index, net_edge_weights, dag_feat, dag_edge_index, dag_edge_weights, Wn0, bn0, Wn1, bn1, Wd0, bd0, Wd1, bd1, Wh1, bh1, Wh2, bh2):
    raise NotImplementedError("write your pallas kernel here")


# speedup vs baseline: 8.8198x; 8.8198x over previous
import functools

import jax
import jax.numpy as jnp
from jax import lax
from jax.experimental import pallas as pl
from jax.experimental.pallas import tpu as pltpu
from jax.experimental.pallas import tpu_sc as plsc

N_NET = 10000
E_NET = 320000
H = 128
N_DAG = 200
E_DAG = 800
D_DAG = 64
ACT_LEN, ACT_DIM = 200, 8

NP = 10240
BLK = 1024
GRID = NP // BLK

try:
    _INFO = plsc.get_sparse_core_info()
    NC, NS = _INFO.num_cores, _INFO.num_subcores
except Exception:
    NC, NS = 2, 16
NW = NC * NS
K = 128
CH = -(-E_NET // (NW * K))
E_PAD = NW * K * CH
ROWS_PER_TILE = NP // NS
F32 = jnp.float32


def _deg_body(dst_hbm, ew_hbm, out_hbm, dstb, ewb, degb, deg_sh, sem):
    c = lax.axis_index("c")
    s = lax.axis_index("s")
    w = c * NS + s
    for i in range(ROWS_PER_TILE // 16):
        degb[pl.ds(i * 16, 16)] = jnp.zeros((16,), F32)
    pltpu.sync_copy(degb, deg_sh.at[pl.ds(s * ROWS_PER_TILE, ROWS_PER_TILE)])
    plsc.subcore_barrier()

    def chunk(i, _):
        off = (w * CH + i) * K
        pltpu.sync_copy(dst_hbm.at[pl.ds(off, K)], dstb)
        pltpu.sync_copy(ew_hbm.at[pl.ds(off, K)], ewb)
        pltpu.sync_copy(ewb, deg_sh.at[dstb], add=True)
        return 0

    lax.fori_loop(0, CH, chunk, 0)
    plsc.subcore_barrier()
    pltpu.sync_copy(deg_sh.at[pl.ds(s * ROWS_PER_TILE, ROWS_PER_TILE)], degb)
    pltpu.sync_copy(degb, out_hbm.at[c, pl.ds(s * ROWS_PER_TILE, ROWS_PER_TILE)])


@jax.jit
def _sc_degree(dst_pad, ew_pad):
    mesh = plsc.VectorSubcoreMesh(core_axis_name="c", subcore_axis_name="s")
    k = pl.kernel(
        _deg_body,
        out_type=jax.ShapeDtypeStruct((NC, NP), F32),
        mesh=mesh,
        scratch_types=[
            pltpu.VMEM((K,), jnp.int32),
            pltpu.VMEM((K,), F32),
            pltpu.VMEM((ROWS_PER_TILE,), F32),
            pltpu.VMEM_SHARED((NP,), F32),
            pltpu.SemaphoreType.DMA,
        ],
    )
    return k(dst_pad, ew_pad)


def _scat_body(z_hbm, src_hbm, dst_hbm, ew_hbm, out_hbm,
               srcb, dstb, ewb, rows, acc_sh, sem):
    c = lax.axis_index("c")
    s = lax.axis_index("s")
    w = c * NS + s

    def zrow(i, _):
        for j in range(H // 16):
            rows[i, pl.ds(j * 16, 16)] = jnp.zeros((16,), F32)
        return 0

    lax.fori_loop(0, K, zrow, 0)

    def zcp(i, _):
        pltpu.sync_copy(rows, acc_sh.at[pl.ds(s * ROWS_PER_TILE + i * K, K)])
        return 0

    lax.fori_loop(0, ROWS_PER_TILE // K, zcp, 0)
    plsc.subcore_barrier()

    def chunk(i, _):
        off = (w * CH + i) * K
        pltpu.sync_copy(src_hbm.at[pl.ds(off, K)], srcb)
        pltpu.sync_copy(dst_hbm.at[pl.ds(off, K)], dstb)
        pltpu.sync_copy(ew_hbm.at[pl.ds(off, K)], ewb)
        pltpu.async_copy(z_hbm.at[srcb], rows, sem).wait()

        def scale(g, _):
            ew16 = ewb[pl.ds(g * 16, 16)]
            for r16 in range(16):
                r = g * 16 + r16
                sv = jnp.full((16,), ew16[r16], F32)
                for j in range(H // 16):
                    rows[r, pl.ds(j * 16, 16)] = rows[r, pl.ds(j * 16, 16)] * sv
            return 0

        lax.fori_loop(0, K // 16, scale, 0)
        pltpu.sync_copy(rows, acc_sh.at[dstb], add=True)
        return 0

    lax.fori_loop(0, CH, chunk, 0)
    plsc.subcore_barrier()

    def ocp(i, _):
        base = s * ROWS_PER_TILE + i * K
        pltpu.sync_copy(acc_sh.at[pl.ds(base, K)], rows)
        pltpu.sync_copy(rows, out_hbm.at[c, pl.ds(base, K)])
        return 0

    lax.fori_loop(0, ROWS_PER_TILE // K, ocp, 0)


@jax.jit
def _sc_scatter(z, src_pad, dst_pad, ew_pad):
    mesh = plsc.VectorSubcoreMesh(core_axis_name="c", subcore_axis_name="s")
    k = pl.kernel(
        _scat_body,
        out_type=jax.ShapeDtypeStruct((NC, NP, H), F32),
        mesh=mesh,
        scratch_types=[
            pltpu.VMEM((K,), jnp.int32),
            pltpu.VMEM((K,), jnp.int32),
            pltpu.VMEM((K,), F32),
            pltpu.VMEM((K, H), F32),
            pltpu.VMEM_SHARED((NP, H), F32),
            pltpu.SemaphoreType.DMA,
        ],
    )
    return k(z, src_pad, dst_pad, ew_pad)


def _dis(degp):
    deg = jnp.sum(degp, axis=1, keepdims=True) + 1.0
    return jnp.where(deg > 0, lax.rsqrt(jnp.where(deg > 0, deg, 1.0)), 0.0)


def _prep_body(x_ref, w_ref, degp_ref, z_ref):
    dis = _dis(degp_ref[...])
    xw = jnp.dot(x_ref[...], w_ref[...], preferred_element_type=F32)
    z_ref[...] = xw * dis


@jax.jit
def _tc_prep(net_feat_p, Wn0, degp):
    return pl.pallas_call(
        _prep_body,
        grid=(GRID,),
        in_specs=[
            pl.BlockSpec((BLK, H), lambda i: (i, 0)),
            pl.BlockSpec((H, H), lambda i: (0, 0)),
            pl.BlockSpec((BLK, 2), lambda i: (i, 0)),
        ],
        out_specs=pl.BlockSpec((BLK, H), lambda i: (i, 0)),
        out_shape=jax.ShapeDtypeStruct((NP, H), F32),
    )(net_feat_p, Wn0, degp)


def _mid_body(p_ref, z0_ref, degp_ref, w_ref, b_ref, z1_ref):
    dis = _dis(degp_ref[...])
    acc = p_ref[0] + p_ref[1] + z0_ref[...]
    emb1 = jnp.maximum(dis * acc + b_ref[...], 0.0)
    z1_ref[...] = jnp.dot(emb1, w_ref[...], preferred_element_type=F32) * dis


@jax.jit
def _tc_mid(p, z0, degp, Wn1, bn0):
    return pl.pallas_call(
        _mid_body,
        grid=(GRID,),
        in_specs=[
            pl.BlockSpec((NC, BLK, H), lambda i: (0, i, 0)),
            pl.BlockSpec((BLK, H), lambda i: (i, 0)),
            pl.BlockSpec((BLK, 2), lambda i: (i, 0)),
            pl.BlockSpec((H, H), lambda i: (0, 0)),
            pl.BlockSpec((1, H), lambda i: (0, 0)),
        ],
        out_specs=pl.BlockSpec((BLK, H), lambda i: (i, 0)),
        out_shape=jax.ShapeDtypeStruct((NP, H), F32),
    )(p, z0, degp, Wn1, bn0.reshape(1, H))


def _dag_body(feat_ref, srcc_ref, dstc_ref, ew_ref,
              wd0_ref, bd0_ref, wd1_ref, bd1_ref, out_ref):
    n_iota = lax.broadcasted_iota(jnp.int32, (1, N_DAG), 1)
    S = (srcc_ref[...] == n_iota).astype(F32)
    D = (dstc_ref[...] == n_iota).astype(F32)
    ew = ew_ref[...]
    deg = jnp.dot(D.T, ew, preferred_element_type=F32) + 1.0
    dis = jnp.where(deg > 0, lax.rsqrt(jnp.where(deg > 0, deg, 1.0)), 0.0)
    A = jnp.dot(D.T, S * ew, preferred_element_type=F32)
    eye = (lax.broadcasted_iota(jnp.int32, (N_DAG, N_DAG), 0)
           == lax.broadcasted_iota(jnp.int32, (N_DAG, N_DAG), 1)).astype(F32)
    Afull = A + eye

    xw0 = jnp.dot(feat_ref[...], wd0_ref[...], preferred_element_type=F32)
    d1 = jnp.maximum(
        dis * jnp.dot(Afull, dis * xw0, preferred_element_type=F32)
        + bd0_ref[...], 0.0)
    xw1 = jnp.dot(d1, wd1_ref[...], preferred_element_type=F32)
    d2 = jnp.maximum(
        dis * jnp.dot(Afull, dis * xw1, preferred_element_type=F32)
        + bd1_ref[...], 0.0)
    ss = jnp.sum(d2 * d2, axis=1, keepdims=True)
    nrm = d2 / jnp.maximum(jnp.sqrt(ss), 1e-12)
    out_ref[...] = jnp.sum(nrm, axis=0, keepdims=True) / N_DAG


@jax.jit
def _tc_dag(dag_feat, dag_src, dag_dst, dag_ew, Wd0, bd0, Wd1, bd1):
    return pl.pallas_call(
        _dag_body,
        out_shape=jax.ShapeDtypeStruct((1, H), F32),
    )(dag_feat, dag_src.reshape(E_DAG, 1), dag_dst.reshape(E_DAG, 1),
      dag_ew.reshape(E_DAG, 1), Wd0, bd0.reshape(1, H), Wd1, bd1.reshape(1, H))


def _fin_body(p_ref, z1_ref, degp_ref, bn1_ref, dagv_ref,
              wh1_ref, bh1_ref, wh2_ref, bh2_ref, out_ref, accs):
    i = pl.program_id(0)

    @pl.when(i == 0)
    def _():
        accs[...] = jnp.zeros_like(accs)

    dis = _dis(degp_ref[...])
    acc = p_ref[0] + p_ref[1] + z1_ref[...]
    emb2 = jnp.maximum(dis * acc + bn1_ref[...], 0.0)
    ss = jnp.sum(emb2 * emb2, axis=1, keepdims=True)
    nrm = emb2 / jnp.maximum(jnp.sqrt(ss), 1e-12)
    rowid = lax.broadcasted_iota(jnp.int32, (BLK, 1), 0) + i * BLK
    nrm = jnp.where(rowid < N_NET, nrm, 0.0)
    accs[...] += jnp.sum(nrm, axis=0, keepdims=True)

    @pl.when(i == pl.num_programs(0) - 1)
    def _():
        hyb = accs[...] / N_NET + dagv_ref[...]
        h = jnp.maximum(
            jnp.dot(hyb, wh1_ref[...], preferred_element_type=F32)
            + bh1_ref[...], 0.0)
        out_ref[...] = (jnp.dot(h, wh2_ref[...], preferred_element_type=F32)
                        + bh2_ref[...])


@jax.jit
def _tc_final(p, z1, degp, bn1, dagv, Wh1, bh1, Wh2, bh2):
    nout = ACT_LEN * ACT_DIM
    return pl.pallas_call(
        _fin_body,
        grid=(GRID,),
        in_specs=[
            pl.BlockSpec((NC, BLK, H), lambda i: (0, i, 0)),
            pl.BlockSpec((BLK, H), lambda i: (i, 0)),
            pl.BlockSpec((BLK, 2), lambda i: (i, 0)),
            pl.BlockSpec((1, H), lambda i: (0, 0)),
            pl.BlockSpec((1, H), lambda i: (0, 0)),
            pl.BlockSpec((H, 256), lambda i: (0, 0)),
            pl.BlockSpec((1, 256), lambda i: (0, 0)),
            pl.BlockSpec((256, nout), lambda i: (0, 0)),
            pl.BlockSpec((1, nout), lambda i: (0, 0)),
        ],
        out_specs=pl.BlockSpec((1, nout), lambda i: (0, 0)),
        out_shape=jax.ShapeDtypeStruct((1, nout), F32),
        scratch_shapes=[pltpu.VMEM((1, H), F32)],
    )(p, z1, degp, bn1.reshape(1, H), dagv, Wh1, bh1.reshape(1, 256),
      Wh2, bh2.reshape(1, nout))


def kernel(net_feat, net_edge_index, net_edge_weights,
           dag_feat, dag_edge_index, dag_edge_weights,
           Wn0, bn0, Wn1, bn1, Wd0, bd0, Wd1, bd1, Wh1, bh1, Wh2, bh2):
    xp = jnp.pad(net_feat, ((0, NP - N_NET), (0, 0)))
    epad = E_PAD - E_NET
    src_pad = jnp.pad(net_edge_index[0], (0, epad))
    dst_pad = jnp.pad(net_edge_index[1], (0, epad))
    ew_pad = jnp.pad(net_edge_weights, (0, epad))

    degp = _sc_degree(dst_pad, ew_pad)
    degp = jnp.transpose(degp)

    dagv = _tc_dag(dag_feat, dag_edge_index[0], dag_edge_index[1],
                   dag_edge_weights, Wd0, bd0, Wd1, bd1)

    z0 = _tc_prep(xp, Wn0, degp)
    p0 = _sc_scatter(z0, src_pad, dst_pad, ew_pad)
    z1 = _tc_mid(p0, z0, degp, Wn1, bn0)
    p1 = _sc_scatter(z1, src_pad, dst_pad, ew_pad)
    act = _tc_final(p1, z1, degp, bn1, dagv, Wh1, bh1, Wh2, bh2)
    return act.reshape(ACT_LEN, ACT_DIM)
